# baseline (speedup 1.0000x reference)
import jax
import jax.numpy as jnp
from jax import lax
from jax.experimental import pallas as pl
from jax.experimental.pallas import tpu as pltpu
from jax.experimental.pallas import tpu_sc as plsc

NUM_CORES = 2
NUM_SUBCORES = 16
NUM_WORKERS = NUM_CORES * NUM_SUBCORES

BATCH = 16384
HIST = 50
EMBED_DIM = 64
XROWS_PER_WORKER = BATCH // NUM_WORKERS
NBUF = 8
N_STEPS = XROWS_PER_WORKER


def _emb_kernel(x_hbm, table_hbm, out_hbm, idx_all, *scratch):
    rows = scratch[:NBUF]
    gsem = scratch[NBUF:2 * NBUF]
    wsem = scratch[2 * NBUF:3 * NBUF]

    wid = lax.axis_index("s") * NUM_CORES + lax.axis_index("c")
    xrow0 = wid * XROWS_PER_WORKER

    pltpu.sync_copy(x_hbm.at[pl.ds(xrow0, XROWS_PER_WORKER), :], idx_all)

    def idx_slice(r):
        return idx_all.at[r, :]

    def out_slice(r):
        return out_hbm.at[xrow0 + r, :, :]

    for b in range(NBUF):
        pltpu.async_copy(table_hbm.at[idx_slice(b)], rows[b], gsem[b])

    @pl.loop(0, N_STEPS, step=NBUF)
    def _round(r0):
        for b in range(NBUF):
            r = r0 + b
            pltpu.make_async_copy(table_hbm.at[idx_slice(r)], rows[b],
                                  gsem[b]).wait()
            pltpu.async_copy(rows[b], out_slice(r), wsem[b])
            nxt = r + NBUF

            @pl.when(nxt < N_STEPS)
            def _refill():
                pltpu.make_async_copy(rows[b], out_slice(r), wsem[b]).wait()
                pltpu.async_copy(table_hbm.at[idx_slice(nxt)], rows[b],
                                 gsem[b])

    for b in range(NBUF):
        r_last = N_STEPS - NBUF + b
        pltpu.make_async_copy(rows[b], out_slice(r_last), wsem[b]).wait()


@jax.jit
def _embedding_lookup(idx, w):
    mesh = plsc.VectorSubcoreMesh(core_axis_name="c", subcore_axis_name="s")
    run = pl.kernel(
        _emb_kernel,
        out_type=jax.ShapeDtypeStruct((BATCH, HIST, EMBED_DIM), jnp.float32),
        mesh=mesh,
        scratch_types=(
            [pltpu.VMEM((XROWS_PER_WORKER, HIST), jnp.int32)]
            + [pltpu.VMEM((HIST, EMBED_DIM), jnp.float32)
               for _ in range(NBUF)]
            + [pltpu.SemaphoreType.DMA for _ in range(2 * NBUF)]
        ),
        compiler_params=pltpu.CompilerParams(use_tc_tiling_on_sc=False),
    )
    return run(idx, w)


def kernel(x, w):
    return _embedding_lookup(x.astype(jnp.int32), w)


# speedup vs baseline: 1.5231x; 1.5231x over previous
import jax
import jax.numpy as jnp
from jax import lax
from jax.experimental import pallas as pl
from jax.experimental.pallas import tpu as pltpu
from jax.experimental.pallas import tpu_sc as plsc

NUM_CORES = 2
NUM_SUBCORES = 16
NUM_WORKERS = NUM_CORES * NUM_SUBCORES

BATCH = 16384
HIST = 50
EMBED_DIM = 64
VOCAB = 1000000
ROW_PAD = 128
XROWS_PER_WORKER = BATCH // NUM_WORKERS
NBUF = 8
N_STEPS = XROWS_PER_WORKER

_VBLK = 4096


def _table_body(wt_ref, o_ref):
    o_ref[:, pl.ds(0, EMBED_DIM)] = jnp.transpose(wt_ref[...])


def _build_table(wt):
    grid = (pl.cdiv(VOCAB, _VBLK),)
    return pl.pallas_call(
        _table_body,
        grid=grid,
        in_specs=[pl.BlockSpec((EMBED_DIM, _VBLK), lambda i: (0, i))],
        out_specs=pl.BlockSpec((_VBLK, ROW_PAD), lambda i: (i, 0)),
        out_shape=jax.ShapeDtypeStruct((VOCAB, ROW_PAD), jnp.float32),
    )(wt)


def _emb_kernel(x_hbm, table_hbm, out_hbm, idx_all, *scratch):
    rows = scratch[:NBUF]
    gsem = scratch[NBUF:2 * NBUF]
    wsem = scratch[2 * NBUF:3 * NBUF]

    wid = lax.axis_index("s") * NUM_CORES + lax.axis_index("c")
    xrow0 = wid * XROWS_PER_WORKER

    pltpu.sync_copy(x_hbm.at[pl.ds(xrow0, XROWS_PER_WORKER), :], idx_all)

    def idx_slice(r):
        return idx_all.at[r, :]

    def out_slice(r):
        return out_hbm.at[xrow0 + r, :, :]

    for b in range(NBUF):
        pltpu.async_copy(table_hbm.at[idx_slice(b)], rows[b], gsem[b])

    @pl.loop(0, N_STEPS, step=NBUF)
    def _round(r0):
        for b in range(NBUF):
            r = r0 + b
            pltpu.make_async_copy(table_hbm.at[idx_slice(r)], rows[b],
                                  gsem[b]).wait()
            pltpu.async_copy(rows[b], out_slice(r), wsem[b])
            nxt = r + NBUF

            @pl.when(nxt < N_STEPS)
            def _refill():
                pltpu.make_async_copy(rows[b], out_slice(r), wsem[b]).wait()
                pltpu.async_copy(table_hbm.at[idx_slice(nxt)], rows[b],
                                 gsem[b])

    for b in range(NBUF):
        r_last = N_STEPS - NBUF + b
        pltpu.make_async_copy(rows[b], out_slice(r_last), wsem[b]).wait()


@jax.jit
def _embedding_lookup(idx, table):
    mesh = plsc.VectorSubcoreMesh(core_axis_name="c", subcore_axis_name="s")
    run = pl.kernel(
        _emb_kernel,
        out_type=jax.ShapeDtypeStruct((BATCH, HIST, ROW_PAD), jnp.float32),
        mesh=mesh,
        scratch_types=(
            [pltpu.VMEM((XROWS_PER_WORKER, HIST), jnp.int32)]
            + [pltpu.VMEM((HIST, ROW_PAD), jnp.float32)
               for _ in range(NBUF)]
            + [pltpu.SemaphoreType.DMA for _ in range(2 * NBUF)]
        ),
        compiler_params=pltpu.CompilerParams(use_tc_tiling_on_sc=True),
    )
    return run(idx, table)


def kernel(x, w):
    table = _build_table(w.T)
    o128 = _embedding_lookup(x.astype(jnp.int32), table)
    return o128[:, :, :EMBED_DIM]
